# baseline (speedup 1.0000x reference)
import functools

import jax
import jax.numpy as jnp
from jax.experimental import pallas as pl
from jax.experimental.pallas import tpu as pltpu

N = 10000
E = 160000
F_NODE = 256
F_BOND = 16
D = 512
LAYERS = 4
F_IN = F_NODE + F_BOND


def _erf(x):
    a1, a2, a3, a4, a5 = (0.254829592, -0.284496736, 1.421413741,
                          -1.453152027, 1.061405429)
    p = 0.3275911
    ax = jnp.abs(x)
    t = 1.0 / (1.0 + p * ax)
    poly = t * (a1 + t * (a2 + t * (a3 + t * (a4 + t * a5))))
    y = 1.0 - poly * jnp.exp(-ax * ax)
    return jnp.sign(x) * y


def _gelu(x):
    return 0.5 * x * (1.0 + _erf(x * 0.7071067811865476))


def _mm_act_kernel(x_ref, w_ref, b_ref, o_ref, *, act):
    acc = jnp.dot(x_ref[...], w_ref[...], preferred_element_type=jnp.float32)
    acc = acc + b_ref[...]
    o_ref[...] = act(acc)


def _mm_act(x, w, b, act, block_r=512):
    R, K = x.shape
    grid = (pl.cdiv(R, block_r),)
    return pl.pallas_call(
        functools.partial(_mm_act_kernel, act=act),
        grid=grid,
        in_specs=[
            pl.BlockSpec((block_r, K), lambda i: (i, 0)),
            pl.BlockSpec((K, D), lambda i: (0, 0)),
            pl.BlockSpec((1, D), lambda i: (0, 0)),
        ],
        out_specs=pl.BlockSpec((block_r, D), lambda i: (i, 0)),
        out_shape=jax.ShapeDtypeStruct((R, D), jnp.float32),
    )(x, w, b.reshape(1, D))


def _rm_kernel(g_ref, ar_ref, w_ref, o_ref):
    g = g_ref[...]
    acc = jnp.dot(g, w_ref[...], preferred_element_type=jnp.float32)
    r = jax.nn.sigmoid(ar_ref[...] + acc)
    o_ref[...] = r * g


def _rm_stage(g, a_r, wr2, block_r=512):
    grid = (pl.cdiv(E, block_r),)
    return pl.pallas_call(
        _rm_kernel,
        grid=grid,
        in_specs=[
            pl.BlockSpec((block_r, D), lambda i: (i, 0)),
            pl.BlockSpec((block_r, D), lambda i: (i, 0)),
            pl.BlockSpec((D, D), lambda i: (0, 0)),
        ],
        out_specs=pl.BlockSpec((block_r, D), lambda i: (i, 0)),
        out_shape=jax.ShapeDtypeStruct((E, D), jnp.float32),
    )(g, a_r, wr2)


def _mess_kernel(s_ref, rij_ref, az_ref, am_ref, wz2_ref, u_ref, o_ref):
    s = s_ref[...]
    z = jax.nn.sigmoid(
        az_ref[...] + jnp.dot(s, wz2_ref[...], preferred_element_type=jnp.float32))
    m = jnp.tanh(
        am_ref[...] + jnp.dot(rij_ref[...], u_ref[...],
                              preferred_element_type=jnp.float32))
    o_ref[...] = (1.0 - z) * s + z * m


def _mess_stage(s, r_ij, a_z, a_m, wz2, u_w, block_r=512):
    grid = (pl.cdiv(E, block_r),)
    return pl.pallas_call(
        _mess_kernel,
        grid=grid,
        in_specs=[
            pl.BlockSpec((block_r, D), lambda i: (i, 0)),
            pl.BlockSpec((block_r, D), lambda i: (i, 0)),
            pl.BlockSpec((block_r, D), lambda i: (i, 0)),
            pl.BlockSpec((block_r, D), lambda i: (i, 0)),
            pl.BlockSpec((D, D), lambda i: (0, 0)),
            pl.BlockSpec((D, D), lambda i: (0, 0)),
        ],
        out_specs=pl.BlockSpec((block_r, D), lambda i: (i, 0)),
        out_shape=jax.ShapeDtypeStruct((E, D), jnp.float32),
    )(s, r_ij, a_z, a_m, wz2, u_w)


def _fbond_kernel(mess_ref, ab_ref, w2_ref, o_ref):
    acc = jnp.dot(mess_ref[...], w2_ref[...], preferred_element_type=jnp.float32)
    o_ref[...] = _gelu(ab_ref[...] + acc)


def _fbond_stage(mess, a_b, w2, block_r=512):
    grid = (pl.cdiv(E, block_r),)
    return pl.pallas_call(
        _fbond_kernel,
        grid=grid,
        in_specs=[
            pl.BlockSpec((block_r, D), lambda i: (i, 0)),
            pl.BlockSpec((block_r, D), lambda i: (i, 0)),
            pl.BlockSpec((D, D), lambda i: (0, 0)),
        ],
        out_specs=pl.BlockSpec((block_r, D), lambda i: (i, 0)),
        out_shape=jax.ShapeDtypeStruct((E, D), jnp.float32),
    )(mess, a_b, w2)


def _fnode_kernel(mn_ref, an_ref, w2_ref, o_ref):
    acc = jnp.dot(mn_ref[...], w2_ref[...], preferred_element_type=jnp.float32)
    o_ref[...] = _gelu(an_ref[...] + acc)


def _fnode_stage(mess_n, a_n, w2, block_r=400):
    grid = (pl.cdiv(N, block_r),)
    return pl.pallas_call(
        _fnode_kernel,
        grid=grid,
        in_specs=[
            pl.BlockSpec((block_r, D), lambda i: (i, 0)),
            pl.BlockSpec((block_r, D), lambda i: (i, 0)),
            pl.BlockSpec((D, D), lambda i: (0, 0)),
        ],
        out_specs=pl.BlockSpec((block_r, D), lambda i: (i, 0)),
        out_shape=jax.ShapeDtypeStruct((N, D), jnp.float32),
    )(mess_n, a_n, w2)


def kernel(node, connect, bond, bond_neighbour,
           W_in_w, W_in_b, Wz_w, Wz_b, Wr_w, Wr_b, U_w, W_w, W_b,
           W_oe_w, W_oe_b, W_on_w, W_on_b):
    src = connect[0]
    tgt = connect[1]
    src_idx = bond_neighbour[0]
    nei_idx = bond_neighbour[1]

    f_ij = jnp.concatenate([jnp.take(node, src, axis=0), bond], axis=-1)

    mess = _mm_act(f_ij, W_in_w, W_in_b, _gelu)

    f_ki = jnp.take(f_ij, src_idx, axis=0)
    a_z = _mm_act(f_ij, Wz_w[:F_IN], Wz_b, lambda x: x)
    a_r = _mm_act(f_ki, Wr_w[:F_IN], Wr_b, lambda x: x)
    a_m = _mm_act(f_ij, W_w, W_b, lambda x: x)
    wz2 = Wz_w[F_IN:]
    wr2 = Wr_w[F_IN:]

    for _ in range(LAYERS):
        g = jnp.take(mess, nei_idx, axis=0)
        s = jax.ops.segment_sum(g, src_idx, num_segments=E)
        rm = _rm_stage(g, a_r, wr2)
        r_ij = jax.ops.segment_sum(rm, src_idx, num_segments=E)
        mess = _mess_stage(s, r_ij, a_z, a_m, wz2, U_w)

    a_b = _mm_act(bond, W_oe_w[:F_BOND], W_oe_b, lambda x: x, block_r=512)
    f_bond = _fbond_stage(mess, a_b, W_oe_w[F_BOND:])

    mess_boost = jax.ops.segment_max(mess, tgt, num_segments=N)
    mess_boost = jnp.where(jnp.isneginf(mess_boost), 0.0, mess_boost)
    mess_sum = jax.ops.segment_sum(mess, tgt, num_segments=N)
    mess_n = mess_sum * mess_boost

    a_n = _mm_act(node, W_on_w[:F_NODE], W_on_b, lambda x: x, block_r=400)
    f_node = _fnode_stage(mess_n, a_n, W_on_w[F_NODE:])
    return (f_node, f_bond)
